# Initial kernel scaffold; baseline (speedup 1.0000x reference)
import jax
import jax.numpy as jnp
from jax.experimental import pallas as pl


def kernel(geometry_points, surface_points, volume_points, geo_tokens, surf_tokens, vol_tokens, params):
    raise NotImplementedError("write your pallas kernel here")


# speedup vs baseline: 4.2304x; 4.2304x over previous
import functools

import jax
import jax.numpy as jnp
from jax import lax
from jax.experimental import pallas as pl
from jax.experimental.pallas import tpu as pltpu
from jax.experimental.pallas import tpu_sc as plsc

N = 2048
HID = 1024
BQH = 64
K1, K2 = 16, 32
P1, P2 = 32, 48
R1SQ = 0.1 * 0.1
R2SQ = 0.2 * 0.2

_STACK_NAMES = ["geo_self", "surf_self", "vol_self", "geo_surf", "geo_vol",
                "surf_geo", "vol_geo"]
_STACK_PAIRS = [(0, 0), (1, 1), (2, 2), (0, 1), (0, 2), (1, 0), (2, 0)]
NST = len(_STACK_PAIRS)

_NW = 32
_QPW = N // _NW


def _bq_sc_body(pts_hbm, out1, out2, pts_v,
                b1x, b1y, b1z, b1v, b2x, b2y, b2z, b2v):
    wid = lax.axis_index("s") * 2 + lax.axis_index("c")
    q0 = wid * _QPW
    pltpu.sync_copy(pts_hbm, pts_v)
    zeros = jnp.zeros((16,), jnp.float32)
    ones = jnp.ones((16,), jnp.float32)

    def zfill(i, _):
        for b in (b1x, b1y, b1z, b1v):
            b[pl.ds(i * 16, 16)] = zeros
        return 0

    def zfill2(i, _):
        for b in (b2x, b2y, b2z, b2v):
            b[pl.ds(i * 16, 16)] = zeros
        return 0

    lax.fori_loop(0, _QPW * P1 // 16, zfill, 0)
    lax.fori_loop(0, _QPW * P2 // 16, zfill2, 0)

    for st, (qset, sset) in enumerate(_STACK_PAIRS):
        qb = 3 * qset * N
        sb = 3 * sset * N

        if st > 0:
            def vfill(i, _):
                b1v[pl.ds(i * 16, 16)] = zeros
                return 0

            def vfill2(i, _):
                b2v[pl.ds(i * 16, 16)] = zeros
                return 0

            lax.fori_loop(0, _QPW * P1 // 16, vfill, 0)
            lax.fori_loop(0, _QPW * P2 // 16, vfill2, 0)

        def per_query(q, carry_unused, qb=qb, sb=sb):
            qg = q0 + (q // 16) * 16
            lanev = jnp.full((16,), q % 16, jnp.int32)
            qx = pts_v[pl.ds(qb + qg, 16)].at[lanev].get(
                mode="promise_in_bounds")
            qy = pts_v[pl.ds(qb + N + qg, 16)].at[lanev].get(
                mode="promise_in_bounds")
            qz = pts_v[pl.ds(qb + 2 * N + qg, 16)].at[lanev].get(
                mode="promise_in_bounds")

            def per_chunk(ch, carry, sb=sb):
                base = ch * 16
                sx = pts_v[pl.ds(sb + base, 16)]
                sy = pts_v[pl.ds(sb + N + base, 16)]
                sz = pts_v[pl.ds(sb + 2 * N + base, 16)]
                dx = sx - qx
                dy = sy - qy
                dz = sz - qz
                d2 = dx * dx + dy * dy + dz * dz
                w2 = d2 <= R2SQ

                def active(carry):
                    c1, c2 = carry

                    def do2(c):
                        pc = jnp.sum(w2.astype(jnp.int32))
                        o = q * P2 + c
                        plsc.store_compressed(b2x.at[pl.ds(o, 16)], dx, mask=w2)
                        plsc.store_compressed(b2y.at[pl.ds(o, 16)], dy, mask=w2)
                        plsc.store_compressed(b2z.at[pl.ds(o, 16)], dz, mask=w2)
                        plsc.store_compressed(b2v.at[pl.ds(o, 16)], ones, mask=w2)
                        return c + pc

                    c2 = lax.cond(c2 < K2, do2, lambda c: c, c2)

                    w1 = d2 <= R1SQ

                    def do1(c):
                        pc = jnp.sum(w1.astype(jnp.int32))
                        o = q * P1 + c
                        plsc.store_compressed(b1x.at[pl.ds(o, 16)], dx, mask=w1)
                        plsc.store_compressed(b1y.at[pl.ds(o, 16)], dy, mask=w1)
                        plsc.store_compressed(b1z.at[pl.ds(o, 16)], dz, mask=w1)
                        plsc.store_compressed(b1v.at[pl.ds(o, 16)], ones, mask=w1)
                        return c + pc

                    c1 = lax.cond(jnp.logical_and(jnp.any(w1), c1 < K1), do1,
                                  lambda c: c, c1)
                    return (c1, c2)

                return lax.cond(jnp.any(w2), active, lambda c: c, carry)

            lax.fori_loop(0, N // 16, per_chunk, (jnp.int32(0), jnp.int32(0)))
            b1v[pl.ds(q * P1 + K1, 16)] = zeros
            b2v[pl.ds(q * P2 + K2, 16)] = zeros
            return carry_unused

        lax.fori_loop(0, _QPW, per_query, jnp.int32(0))
        for ch, b in enumerate((b1x, b1y, b1z, b1v)):
            pltpu.sync_copy(b, out1.at[pl.ds(((st * 4 + ch) * N + q0) * P1,
                                             _QPW * P1)])
        for ch, b in enumerate((b2x, b2y, b2z, b2v)):
            pltpu.sync_copy(b, out2.at[pl.ds(((st * 4 + ch) * N + q0) * P2,
                                             _QPW * P2)])


@functools.lru_cache(maxsize=1)
def _bq_sc_build():
    return pl.kernel(
        _bq_sc_body,
        out_type=(
            jax.ShapeDtypeStruct((NST * 4 * N * P1,), jnp.float32),
            jax.ShapeDtypeStruct((NST * 4 * N * P2,), jnp.float32),
        ),
        mesh=plsc.VectorSubcoreMesh(core_axis_name="c", subcore_axis_name="s"),
        compiler_params=pltpu.CompilerParams(needs_layout_passes=False),
        scratch_types=[
            pltpu.VMEM((3 * 3 * N,), jnp.float32),
            pltpu.VMEM((_QPW * P1,), jnp.float32),
            pltpu.VMEM((_QPW * P1,), jnp.float32),
            pltpu.VMEM((_QPW * P1,), jnp.float32),
            pltpu.VMEM((_QPW * P1,), jnp.float32),
            pltpu.VMEM((_QPW * P2,), jnp.float32),
            pltpu.VMEM((_QPW * P2,), jnp.float32),
            pltpu.VMEM((_QPW * P2,), jnp.float32),
            pltpu.VMEM((_QPW * P2,), jnp.float32),
        ],
    )


_QB = 256
_QSET = [qs for qs, _ in _STACK_PAIRS]


def _mlp_tc_body(rel1, rel2, tg, ts, tv, w1, b1, w2, b2, wo, bo,
                 og, osf, ov):
    st = pl.program_id(1)
    feats = []
    for sc, P in enumerate((P1, P2)):
        rel = (rel1 if sc == 0 else rel2)[0]
        h = jnp.dot(rel, w1[0, sc], preferred_element_type=jnp.float32)
        h = jnp.maximum(h + b1[0, sc][None, :], 0.0)
        h = jnp.dot(h, w2[0, sc], preferred_element_type=jnp.float32)
        h = jnp.maximum(h + b2[0, sc][None, :], 0.0)
        v = jnp.broadcast_to(rel[:, 3:4], h.shape)
        h = (h * v).reshape(_QB, P, BQH)
        feats.append(jnp.max(h, axis=1))
    f = jnp.concatenate(feats, axis=-1)
    contrib = jnp.dot(f, wo[0], preferred_element_type=jnp.float32)
    contrib = contrib + bo[0]

    for out_ref, tok_ref, qset in ((og, tg, 0), (osf, ts, 1), (ov, tv, 2)):
        first = [s for s, q in enumerate(_QSET) if q == qset][0]
        rest = [s for s, q in enumerate(_QSET) if q == qset][1:]

        @pl.when(st == first)
        def _():
            out_ref[...] = tok_ref[...] + contrib

        for s in rest:
            @pl.when(st == s)
            def _():
                out_ref[...] = out_ref[...] + contrib


def _mlp_tc(rel4_1, rel4_2, tg, ts, tv, W1e, b1s, W2s, b2s, WOs, BOs):
    grid = (N // _QB, NST)
    tok_spec = pl.BlockSpec((_QB, HID), lambda i, st: (i, 0))
    return pl.pallas_call(
        _mlp_tc_body,
        grid=grid,
        in_specs=[
            pl.BlockSpec((1, _QB * P1, 4), lambda i, st: (st, i, 0)),
            pl.BlockSpec((1, _QB * P2, 4), lambda i, st: (st, i, 0)),
            tok_spec, tok_spec, tok_spec,
            pl.BlockSpec((1, 2, 4, BQH), lambda i, st: (st, 0, 0, 0)),
            pl.BlockSpec((1, 2, BQH), lambda i, st: (st, 0, 0)),
            pl.BlockSpec((1, 2, BQH, BQH), lambda i, st: (st, 0, 0, 0)),
            pl.BlockSpec((1, 2, BQH), lambda i, st: (st, 0, 0)),
            pl.BlockSpec((1, 2 * BQH, HID), lambda i, st: (st, 0, 0)),
            pl.BlockSpec((1, 1, HID), lambda i, st: (st, 0, 0)),
        ],
        out_specs=[tok_spec, tok_spec, tok_spec],
        out_shape=[jax.ShapeDtypeStruct((N, HID), jnp.float32)] * 3,
    )(rel4_1, rel4_2, tg, ts, tv, W1e, b1s, W2s, b2s,
      WOs, BOs.reshape(NST, 1, HID))


def kernel(geometry_points, surface_points, volume_points, geo_tokens,
           surf_tokens, vol_tokens, params):
    pts = jnp.stack([geometry_points[0].T, surface_points[0].T,
                     volume_points[0].T])
    o1, o2 = _bq_sc_build()(pts.reshape(-1))
    rel4_1 = o1.reshape(NST, 4, N * P1).transpose(0, 2, 1)
    rel4_2 = o2.reshape(NST, 4, N * P2).transpose(0, 2, 1)

    W1s = jnp.stack([jnp.stack([params[n]["scales"][i]["W1"] for i in range(2)])
                     for n in _STACK_NAMES])
    W1e = jnp.concatenate(
        [W1s, jnp.zeros((NST, 2, 1, BQH), jnp.float32)], axis=2)
    b1s = jnp.stack([jnp.stack([params[n]["scales"][i]["b1"] for i in range(2)])
                     for n in _STACK_NAMES])
    W2s = jnp.stack([jnp.stack([params[n]["scales"][i]["W2"] for i in range(2)])
                     for n in _STACK_NAMES])
    b2s = jnp.stack([jnp.stack([params[n]["scales"][i]["b2"] for i in range(2)])
                     for n in _STACK_NAMES])
    WOs = jnp.stack([params[n]["Wout"] for n in _STACK_NAMES])
    BOs = jnp.stack([params[n]["bout"] for n in _STACK_NAMES])

    geo, surf, vol = _mlp_tc(rel4_1, rel4_2, geo_tokens[0], surf_tokens[0],
                             vol_tokens[0], W1e, b1s, W2s, b2s, WOs, BOs)
    return geo[None], surf[None], vol[None]


# speedup vs baseline: 5.6945x; 1.3461x over previous
import functools

import jax
import jax.numpy as jnp
from jax import lax
from jax.experimental import pallas as pl
from jax.experimental.pallas import tpu as pltpu
from jax.experimental.pallas import tpu_sc as plsc

N = 2048
HID = 1024
BQH = 64
K1, K2 = 16, 32
P1, P2 = 32, 48
R1SQ = 0.1 * 0.1
R2SQ = 0.2 * 0.2

_STACK_NAMES = ["geo_self", "surf_self", "vol_self", "geo_surf", "geo_vol",
                "surf_geo", "vol_geo"]
_STACK_PAIRS = [(0, 0), (1, 1), (2, 2), (0, 1), (0, 2), (1, 0), (2, 0)]
NST = len(_STACK_PAIRS)

_NW = 32
_QPW = N // _NW


def _bq_sc_body(pts_hbm, out1, out2, pts_v,
                b1x, b1y, b1z, b1v, b2x, b2y, b2z, b2v):
    wid = lax.axis_index("s") * 2 + lax.axis_index("c")
    q0 = wid * _QPW
    pltpu.sync_copy(pts_hbm, pts_v)
    zeros = jnp.zeros((16,), jnp.float32)
    ones = jnp.ones((16,), jnp.float32)

    def zfill(i, _):
        for b in (b1x, b1y, b1z, b1v):
            b[pl.ds(i * 16, 16)] = zeros
        return 0

    def zfill2(i, _):
        for b in (b2x, b2y, b2z, b2v):
            b[pl.ds(i * 16, 16)] = zeros
        return 0

    lax.fori_loop(0, _QPW * P1 // 16, zfill, 0)
    lax.fori_loop(0, _QPW * P2 // 16, zfill2, 0)

    for st, (qset, sset) in enumerate(_STACK_PAIRS):
        qb = 3 * qset * N
        sb = 3 * sset * N

        if st > 0:
            def vfill(i, _):
                b1v[pl.ds(i * 16, 16)] = zeros
                return 0

            def vfill2(i, _):
                b2v[pl.ds(i * 16, 16)] = zeros
                return 0

            lax.fori_loop(0, _QPW * P1 // 16, vfill, 0)
            lax.fori_loop(0, _QPW * P2 // 16, vfill2, 0)

        def per_query(q, carry_unused, qb=qb, sb=sb):
            qg = q0 + (q // 16) * 16
            lanev = jnp.full((16,), q % 16, jnp.int32)
            qx = pts_v[pl.ds(qb + qg, 16)].at[lanev].get(
                mode="promise_in_bounds")
            qy = pts_v[pl.ds(qb + N + qg, 16)].at[lanev].get(
                mode="promise_in_bounds")
            qz = pts_v[pl.ds(qb + 2 * N + qg, 16)].at[lanev].get(
                mode="promise_in_bounds")

            def per_chunk(ch, carry, sb=sb):
                base = ch * 16
                sx = pts_v[pl.ds(sb + base, 16)]
                sy = pts_v[pl.ds(sb + N + base, 16)]
                sz = pts_v[pl.ds(sb + 2 * N + base, 16)]
                dx = sx - qx
                dy = sy - qy
                dz = sz - qz
                d2 = dx * dx + dy * dy + dz * dz
                w2 = d2 <= R2SQ

                def active(carry):
                    c1, c2 = carry

                    def do2(c):
                        pc = jnp.sum(w2.astype(jnp.int32))
                        o = q * P2 + c
                        plsc.store_compressed(b2x.at[pl.ds(o, 16)], dx, mask=w2)
                        plsc.store_compressed(b2y.at[pl.ds(o, 16)], dy, mask=w2)
                        plsc.store_compressed(b2z.at[pl.ds(o, 16)], dz, mask=w2)
                        plsc.store_compressed(b2v.at[pl.ds(o, 16)], ones, mask=w2)
                        return c + pc

                    c2 = lax.cond(c2 < K2, do2, lambda c: c, c2)

                    w1 = d2 <= R1SQ

                    def do1(c):
                        pc = jnp.sum(w1.astype(jnp.int32))
                        o = q * P1 + c
                        plsc.store_compressed(b1x.at[pl.ds(o, 16)], dx, mask=w1)
                        plsc.store_compressed(b1y.at[pl.ds(o, 16)], dy, mask=w1)
                        plsc.store_compressed(b1z.at[pl.ds(o, 16)], dz, mask=w1)
                        plsc.store_compressed(b1v.at[pl.ds(o, 16)], ones, mask=w1)
                        return c + pc

                    c1 = lax.cond(jnp.logical_and(jnp.any(w1), c1 < K1), do1,
                                  lambda c: c, c1)
                    return (c1, c2)

                return lax.cond(jnp.any(w2), active, lambda c: c, carry)

            lax.fori_loop(0, N // 16, per_chunk, (jnp.int32(0), jnp.int32(0)))
            b1v[pl.ds(q * P1 + K1, 16)] = zeros
            b2v[pl.ds(q * P2 + K2, 16)] = zeros
            return carry_unused

        lax.fori_loop(0, _QPW, per_query, jnp.int32(0))
        for ch, b in enumerate((b1x, b1y, b1z, b1v)):
            pltpu.sync_copy(b, out1.at[pl.ds(((st * 4 + ch) * N + q0) * P1,
                                             _QPW * P1)])
        for ch, b in enumerate((b2x, b2y, b2z, b2v)):
            pltpu.sync_copy(b, out2.at[pl.ds(((st * 4 + ch) * N + q0) * P2,
                                             _QPW * P2)])


@functools.lru_cache(maxsize=1)
def _bq_sc_build():
    return pl.kernel(
        _bq_sc_body,
        out_type=(
            jax.ShapeDtypeStruct((NST * 4 * N * P1,), jnp.float32),
            jax.ShapeDtypeStruct((NST * 4 * N * P2,), jnp.float32),
        ),
        mesh=plsc.VectorSubcoreMesh(core_axis_name="c", subcore_axis_name="s"),
        compiler_params=pltpu.CompilerParams(needs_layout_passes=False),
        scratch_types=[
            pltpu.VMEM((3 * 3 * N,), jnp.float32),
            pltpu.VMEM((_QPW * P1,), jnp.float32),
            pltpu.VMEM((_QPW * P1,), jnp.float32),
            pltpu.VMEM((_QPW * P1,), jnp.float32),
            pltpu.VMEM((_QPW * P1,), jnp.float32),
            pltpu.VMEM((_QPW * P2,), jnp.float32),
            pltpu.VMEM((_QPW * P2,), jnp.float32),
            pltpu.VMEM((_QPW * P2,), jnp.float32),
            pltpu.VMEM((_QPW * P2,), jnp.float32),
        ],
    )


_QB = 256
_QSET = [qs for qs, _ in _STACK_PAIRS]


def _mlp_tc_body(rel1, rel2, tg, ts, tv, w1, b1, w2, b2, wo, bo,
                 og, osf, ov):
    st = pl.program_id(1)
    wv = (lax.broadcasted_iota(jnp.int32, (4, BQH), 0) == 3).astype(jnp.float32)
    feats = []
    for sc, P in enumerate((P1, P2)):
        rel = (rel1 if sc == 0 else rel2)[0]
        dn = (((0,), (0,)), ((), ()))
        h = lax.dot_general(rel, w1[0, sc], dn,
                            preferred_element_type=jnp.float32)
        h = jnp.maximum(h + b1[0, sc][None, :], 0.0)
        h = jnp.dot(h, w2[0, sc], preferred_element_type=jnp.float32)
        h = jnp.maximum(h + b2[0, sc][None, :], 0.0)
        v = lax.dot_general(rel, wv, dn,
                            preferred_element_type=jnp.float32)
        h = (h * v).reshape(_QB, P, BQH)
        feats.append(jnp.max(h, axis=1))
    f = jnp.concatenate(feats, axis=-1)
    contrib = jnp.dot(f, wo[0], preferred_element_type=jnp.float32)
    contrib = contrib + bo[0]

    for out_ref, tok_ref, qset in ((og, tg, 0), (osf, ts, 1), (ov, tv, 2)):
        first = [s for s, q in enumerate(_QSET) if q == qset][0]
        rest = [s for s, q in enumerate(_QSET) if q == qset][1:]

        @pl.when(st == first)
        def _():
            out_ref[...] = tok_ref[...] + contrib

        for s in rest:
            @pl.when(st == s)
            def _():
                out_ref[...] = out_ref[...] + contrib


def _mlp_tc(rel4_1, rel4_2, tg, ts, tv, W1e, b1s, W2s, b2s, WOs, BOs):
    grid = (N // _QB, NST)
    tok_spec = pl.BlockSpec((_QB, HID), lambda i, st: (i, 0))
    return pl.pallas_call(
        _mlp_tc_body,
        grid=grid,
        in_specs=[
            pl.BlockSpec((1, 4, _QB * P1), lambda i, st: (st, 0, i)),
            pl.BlockSpec((1, 4, _QB * P2), lambda i, st: (st, 0, i)),
            tok_spec, tok_spec, tok_spec,
            pl.BlockSpec((1, 2, 4, BQH), lambda i, st: (st, 0, 0, 0)),
            pl.BlockSpec((1, 2, BQH), lambda i, st: (st, 0, 0)),
            pl.BlockSpec((1, 2, BQH, BQH), lambda i, st: (st, 0, 0, 0)),
            pl.BlockSpec((1, 2, BQH), lambda i, st: (st, 0, 0)),
            pl.BlockSpec((1, 2 * BQH, HID), lambda i, st: (st, 0, 0)),
            pl.BlockSpec((1, 1, HID), lambda i, st: (st, 0, 0)),
        ],
        out_specs=[tok_spec, tok_spec, tok_spec],
        out_shape=[jax.ShapeDtypeStruct((N, HID), jnp.float32)] * 3,
    )(rel4_1, rel4_2, tg, ts, tv, W1e, b1s, W2s, b2s,
      WOs, BOs.reshape(NST, 1, HID))


def kernel(geometry_points, surface_points, volume_points, geo_tokens,
           surf_tokens, vol_tokens, params):
    pts = jnp.stack([geometry_points[0].T, surface_points[0].T,
                     volume_points[0].T])
    o1, o2 = _bq_sc_build()(pts.reshape(-1))
    rel4_1 = o1.reshape(NST, 4, N * P1)
    rel4_2 = o2.reshape(NST, 4, N * P2)

    W1s = jnp.stack([jnp.stack([params[n]["scales"][i]["W1"] for i in range(2)])
                     for n in _STACK_NAMES])
    W1e = jnp.concatenate(
        [W1s, jnp.zeros((NST, 2, 1, BQH), jnp.float32)], axis=2)
    b1s = jnp.stack([jnp.stack([params[n]["scales"][i]["b1"] for i in range(2)])
                     for n in _STACK_NAMES])
    W2s = jnp.stack([jnp.stack([params[n]["scales"][i]["W2"] for i in range(2)])
                     for n in _STACK_NAMES])
    b2s = jnp.stack([jnp.stack([params[n]["scales"][i]["b2"] for i in range(2)])
                     for n in _STACK_NAMES])
    WOs = jnp.stack([params[n]["Wout"] for n in _STACK_NAMES])
    BOs = jnp.stack([params[n]["bout"] for n in _STACK_NAMES])

    geo, surf, vol = _mlp_tc(rel4_1, rel4_2, geo_tokens[0], surf_tokens[0],
                             vol_tokens[0], W1e, b1s, W2s, b2s, WOs, BOs)
    return geo[None], surf[None], vol[None]


# speedup vs baseline: 10.1145x; 1.7762x over previous
import functools

import jax
import jax.numpy as jnp
from jax import lax
from jax.experimental import pallas as pl
from jax.experimental.pallas import tpu as pltpu
from jax.experimental.pallas import tpu_sc as plsc

N = 2048
HID = 1024
BQH = 64
K1, K2 = 16, 32
P1, P2 = 32, 48
R1SQ = 0.1 * 0.1
R2SQ = 0.2 * 0.2

_STACK_NAMES = ["geo_self", "surf_self", "vol_self", "geo_surf", "geo_vol",
                "surf_geo", "vol_geo"]
_STACK_PAIRS = [(0, 0), (1, 1), (2, 2), (0, 1), (0, 2), (1, 0), (2, 0)]
NST = len(_STACK_PAIRS)

_NW = 32
_QPW = N // _NW


def _bq_sc_body(pts_hbm, out1, out2, pts_v,
                b1x, b1y, b1z, b1v, b2x, b2y, b2z, b2v):
    wid = lax.axis_index("s") * 2 + lax.axis_index("c")
    q0 = wid * _QPW
    pltpu.sync_copy(pts_hbm, pts_v)
    zeros = jnp.zeros((16,), jnp.float32)
    ones = jnp.ones((16,), jnp.float32)

    def zfill(i, _):
        for b in (b1x, b1y, b1z, b1v):
            b[pl.ds(i * 16, 16)] = zeros
        return 0

    def zfill2(i, _):
        for b in (b2x, b2y, b2z, b2v):
            b[pl.ds(i * 16, 16)] = zeros
        return 0

    lax.fori_loop(0, _QPW * P1 // 16, zfill, 0)
    lax.fori_loop(0, _QPW * P2 // 16, zfill2, 0)

    for st, (qset, sset) in enumerate(_STACK_PAIRS):
        qb = 3 * qset * N
        sb = 3 * sset * N

        if st > 0:
            def vfill(i, _):
                b1v[pl.ds(i * 16, 16)] = zeros
                return 0

            def vfill2(i, _):
                b2v[pl.ds(i * 16, 16)] = zeros
                return 0

            lax.fori_loop(0, _QPW * P1 // 16, vfill, 0)
            lax.fori_loop(0, _QPW * P2 // 16, vfill2, 0)

        def per_query(q, carry_unused, qb=qb, sb=sb):
            qg = q0 + (q // 16) * 16
            lanev = jnp.full((16,), q % 16, jnp.int32)
            qx = pts_v[pl.ds(qb + qg, 16)].at[lanev].get(
                mode="promise_in_bounds")
            qy = pts_v[pl.ds(qb + N + qg, 16)].at[lanev].get(
                mode="promise_in_bounds")
            qz = pts_v[pl.ds(qb + 2 * N + qg, 16)].at[lanev].get(
                mode="promise_in_bounds")

            def per_chunk(ch, carry, sb=sb):
                c1v, c2v = carry
                base = ch * 16
                sx = pts_v[pl.ds(sb + base, 16)]
                sy = pts_v[pl.ds(sb + N + base, 16)]
                sz = pts_v[pl.ds(sb + 2 * N + base, 16)]
                dx = sx - qx
                dy = sy - qy
                dz = sz - qz
                d2 = dx * dx + dy * dy + dz * dz
                w2 = d2 <= R2SQ
                w1 = d2 <= R1SQ
                slot2 = c2v + plsc.cumsum(w2.astype(jnp.int32)) - 1
                slot1 = c1v + plsc.cumsum(w1.astype(jnp.int32)) - 1
                sel2 = jnp.logical_and(w2, slot2 < K2)
                sel1 = jnp.logical_and(w1, slot1 < K1)
                i2 = slot2 + q * P2
                i1 = slot1 + q * P1
                plsc.store_scatter(b2x, [i2], dx, mask=sel2)
                plsc.store_scatter(b2y, [i2], dy, mask=sel2)
                plsc.store_scatter(b2z, [i2], dz, mask=sel2)
                plsc.store_scatter(b2v, [i2], ones, mask=sel2)
                plsc.store_scatter(b1x, [i1], dx, mask=sel1)
                plsc.store_scatter(b1y, [i1], dy, mask=sel1)
                plsc.store_scatter(b1z, [i1], dz, mask=sel1)
                plsc.store_scatter(b1v, [i1], ones, mask=sel1)
                c2v = jnp.minimum(c2v + plsc.all_reduce_population_count(w2),
                                  K2)
                c1v = jnp.minimum(c1v + plsc.all_reduce_population_count(w1),
                                  K1)
                return (c1v, c2v)

            zi = jnp.zeros((16,), jnp.int32)
            lax.fori_loop(0, N // 16, per_chunk, (zi, zi))
            return carry_unused

        lax.fori_loop(0, _QPW, per_query, jnp.int32(0))
        for ch, b in enumerate((b1x, b1y, b1z, b1v)):
            pltpu.sync_copy(b, out1.at[pl.ds(((st * 4 + ch) * N + q0) * P1,
                                             _QPW * P1)])
        for ch, b in enumerate((b2x, b2y, b2z, b2v)):
            pltpu.sync_copy(b, out2.at[pl.ds(((st * 4 + ch) * N + q0) * P2,
                                             _QPW * P2)])


@functools.lru_cache(maxsize=1)
def _bq_sc_build():
    return pl.kernel(
        _bq_sc_body,
        out_type=(
            jax.ShapeDtypeStruct((NST * 4 * N * P1,), jnp.float32),
            jax.ShapeDtypeStruct((NST * 4 * N * P2,), jnp.float32),
        ),
        mesh=plsc.VectorSubcoreMesh(core_axis_name="c", subcore_axis_name="s"),
        compiler_params=pltpu.CompilerParams(needs_layout_passes=False),
        scratch_types=[
            pltpu.VMEM((3 * 3 * N,), jnp.float32),
            pltpu.VMEM((_QPW * P1,), jnp.float32),
            pltpu.VMEM((_QPW * P1,), jnp.float32),
            pltpu.VMEM((_QPW * P1,), jnp.float32),
            pltpu.VMEM((_QPW * P1,), jnp.float32),
            pltpu.VMEM((_QPW * P2,), jnp.float32),
            pltpu.VMEM((_QPW * P2,), jnp.float32),
            pltpu.VMEM((_QPW * P2,), jnp.float32),
            pltpu.VMEM((_QPW * P2,), jnp.float32),
        ],
    )


_QB = 256
_QSET = [qs for qs, _ in _STACK_PAIRS]


def _mlp_tc_body(rel1, rel2, tg, ts, tv, w1, b1, w2, b2, wo, bo,
                 og, osf, ov):
    st = pl.program_id(1)
    wv = (lax.broadcasted_iota(jnp.int32, (4, BQH), 0) == 3).astype(jnp.float32)
    feats = []
    for sc, P in enumerate((P1, P2)):
        rel = (rel1 if sc == 0 else rel2)[0]
        dn = (((0,), (0,)), ((), ()))
        h = lax.dot_general(rel, w1[0, sc], dn,
                            preferred_element_type=jnp.float32)
        h = jnp.maximum(h + b1[0, sc][None, :], 0.0)
        h = jnp.dot(h, w2[0, sc], preferred_element_type=jnp.float32)
        h = jnp.maximum(h + b2[0, sc][None, :], 0.0)
        v = lax.dot_general(rel, wv, dn,
                            preferred_element_type=jnp.float32)
        h = (h * v).reshape(_QB, P, BQH)
        feats.append(jnp.max(h, axis=1))
    f = jnp.concatenate(feats, axis=-1)
    contrib = jnp.dot(f, wo[0], preferred_element_type=jnp.float32)
    contrib = contrib + bo[0]

    for out_ref, tok_ref, qset in ((og, tg, 0), (osf, ts, 1), (ov, tv, 2)):
        first = [s for s, q in enumerate(_QSET) if q == qset][0]
        rest = [s for s, q in enumerate(_QSET) if q == qset][1:]

        @pl.when(st == first)
        def _():
            out_ref[...] = tok_ref[...] + contrib

        for s in rest:
            @pl.when(st == s)
            def _():
                out_ref[...] = out_ref[...] + contrib


def _mlp_tc(rel4_1, rel4_2, tg, ts, tv, W1e, b1s, W2s, b2s, WOs, BOs):
    grid = (N // _QB, NST)
    tok_spec = pl.BlockSpec((_QB, HID), lambda i, st: (i, 0))
    return pl.pallas_call(
        _mlp_tc_body,
        grid=grid,
        in_specs=[
            pl.BlockSpec((1, 4, _QB * P1), lambda i, st: (st, 0, i)),
            pl.BlockSpec((1, 4, _QB * P2), lambda i, st: (st, 0, i)),
            tok_spec, tok_spec, tok_spec,
            pl.BlockSpec((1, 2, 4, BQH), lambda i, st: (st, 0, 0, 0)),
            pl.BlockSpec((1, 2, BQH), lambda i, st: (st, 0, 0)),
            pl.BlockSpec((1, 2, BQH, BQH), lambda i, st: (st, 0, 0, 0)),
            pl.BlockSpec((1, 2, BQH), lambda i, st: (st, 0, 0)),
            pl.BlockSpec((1, 2 * BQH, HID), lambda i, st: (st, 0, 0)),
            pl.BlockSpec((1, 1, HID), lambda i, st: (st, 0, 0)),
        ],
        out_specs=[tok_spec, tok_spec, tok_spec],
        out_shape=[jax.ShapeDtypeStruct((N, HID), jnp.float32)] * 3,
    )(rel4_1, rel4_2, tg, ts, tv, W1e, b1s, W2s, b2s,
      WOs, BOs.reshape(NST, 1, HID))


def kernel(geometry_points, surface_points, volume_points, geo_tokens,
           surf_tokens, vol_tokens, params):
    pts = jnp.stack([geometry_points[0].T, surface_points[0].T,
                     volume_points[0].T])
    o1, o2 = _bq_sc_build()(pts.reshape(-1))
    rel4_1 = o1.reshape(NST, 4, N * P1)
    rel4_2 = o2.reshape(NST, 4, N * P2)

    W1s = jnp.stack([jnp.stack([params[n]["scales"][i]["W1"] for i in range(2)])
                     for n in _STACK_NAMES])
    W1e = jnp.concatenate(
        [W1s, jnp.zeros((NST, 2, 1, BQH), jnp.float32)], axis=2)
    b1s = jnp.stack([jnp.stack([params[n]["scales"][i]["b1"] for i in range(2)])
                     for n in _STACK_NAMES])
    W2s = jnp.stack([jnp.stack([params[n]["scales"][i]["W2"] for i in range(2)])
                     for n in _STACK_NAMES])
    b2s = jnp.stack([jnp.stack([params[n]["scales"][i]["b2"] for i in range(2)])
                     for n in _STACK_NAMES])
    WOs = jnp.stack([params[n]["Wout"] for n in _STACK_NAMES])
    BOs = jnp.stack([params[n]["bout"] for n in _STACK_NAMES])

    geo, surf, vol = _mlp_tc(rel4_1, rel4_2, geo_tokens[0], surf_tokens[0],
                             vol_tokens[0], W1e, b1s, W2s, b2s, WOs, BOs)
    return geo[None], surf[None], vol[None]


# speedup vs baseline: 15.9487x; 1.5768x over previous
import functools

import jax
import jax.numpy as jnp
from jax import lax
from jax.experimental import pallas as pl
from jax.experimental.pallas import tpu as pltpu
from jax.experimental.pallas import tpu_sc as plsc

N = 2048
HID = 1024
BQH = 64
K1, K2 = 16, 32
P1, P2 = 32, 48
R1SQ = 0.1 * 0.1
R2SQ = 0.2 * 0.2

_STACK_NAMES = ["geo_self", "surf_self", "vol_self", "geo_surf", "geo_vol",
                "surf_geo", "vol_geo"]
_STACK_PAIRS = [(0, 0), (1, 1), (2, 2), (0, 1), (0, 2), (1, 0), (2, 0)]
NST = len(_STACK_PAIRS)

_NW = 32
_QPW = N // _NW


def _bq_sc_body(pts_hbm, out1, out2, pts_v,
                b1x, b1y, b1z, b1v, b2x, b2y, b2z, b2v):
    wid = lax.axis_index("s") * 2 + lax.axis_index("c")
    q0 = wid * _QPW
    pltpu.sync_copy(pts_hbm, pts_v)
    zeros = jnp.zeros((16,), jnp.float32)
    lane = lax.iota(jnp.int32, 16)

    def zfill(i, _):
        for b in (b1x, b1y, b1z):
            b[pl.ds(i * 16, 16)] = zeros
        return 0

    def zfill2(i, _):
        for b in (b2x, b2y, b2z):
            b[pl.ds(i * 16, 16)] = zeros
        return 0

    lax.fori_loop(0, _QPW * P1 // 16, zfill, 0)
    lax.fori_loop(0, _QPW * P2 // 16, zfill2, 0)

    def splat(v, j):
        return v.at[jnp.full((16,), j, jnp.int32)].get(mode="promise_in_bounds")

    for st, (qset, sset) in enumerate(_STACK_PAIRS):
        qb = 3 * qset * N
        sb = 3 * sset * N

        def per_pair(qq, carry_unused, qb=qb, sb=sb):
            qa = 2 * qq
            qg = q0 + (qa // 16) * 16
            cx = pts_v[pl.ds(qb + qg, 16)]
            cy = pts_v[pl.ds(qb + N + qg, 16)]
            cz = pts_v[pl.ds(qb + 2 * N + qg, 16)]
            la = qa % 16
            qxa, qya, qza = splat(cx, la), splat(cy, la), splat(cz, la)
            qxb, qyb, qzb = splat(cx, la + 1), splat(cy, la + 1), splat(cz, la + 1)
            base1a, base2a = qa * P1, qa * P2
            base1b, base2b = base1a + P1, base2a + P2
            cap1a = jnp.full((16,), base1a + K1, jnp.int32)
            cap2a = jnp.full((16,), base2a + K2, jnp.int32)
            cap1b = jnp.full((16,), base1b + K1, jnp.int32)
            cap2b = jnp.full((16,), base2b + K2, jnp.int32)

            def per_chunk(ch, carry, sb=sb):
                a1a, a2a, a1b, a2b = carry
                base = ch * 16
                sx = pts_v[pl.ds(sb + base, 16)]
                sy = pts_v[pl.ds(sb + N + base, 16)]
                sz = pts_v[pl.ds(sb + 2 * N + base, 16)]
                out = []
                for (qx, qy, qz, a1, a2, cap1, cap2) in (
                        (qxa, qya, qza, a1a, a2a, cap1a, cap2a),
                        (qxb, qyb, qzb, a1b, a2b, cap1b, cap2b)):
                    dx = sx - qx
                    dy = sy - qy
                    dz = sz - qz
                    d2 = dx * dx + dy * dy + dz * dz
                    w2 = d2 <= R2SQ
                    w1 = d2 <= R1SQ
                    i2 = a2 + plsc.cumsum(w2.astype(jnp.int32))
                    i1 = a1 + plsc.cumsum(w1.astype(jnp.int32))
                    sel2 = jnp.logical_and(w2, i2 < cap2)
                    sel1 = jnp.logical_and(w1, i1 < cap1)
                    plsc.store_scatter(b2x, [i2], dx, mask=sel2)
                    plsc.store_scatter(b2y, [i2], dy, mask=sel2)
                    plsc.store_scatter(b2z, [i2], dz, mask=sel2)
                    plsc.store_scatter(b1x, [i1], dx, mask=sel1)
                    plsc.store_scatter(b1y, [i1], dy, mask=sel1)
                    plsc.store_scatter(b1z, [i1], dz, mask=sel1)
                    a2 = jnp.minimum(
                        a2 + plsc.all_reduce_population_count(w2), cap2 - 1)
                    a1 = jnp.minimum(
                        a1 + plsc.all_reduce_population_count(w1), cap1 - 1)
                    out.append((a1, a2))
                return (out[0][0], out[0][1], out[1][0], out[1][1])

            init = (jnp.full((16,), base1a - 1, jnp.int32),
                    jnp.full((16,), base2a - 1, jnp.int32),
                    jnp.full((16,), base1b - 1, jnp.int32),
                    jnp.full((16,), base2b - 1, jnp.int32))
            a1a, a2a, a1b, a2b = lax.fori_loop(0, N // 16, per_chunk, init)

            for (a1, a2, b1o, b2o) in ((a1a, a2a, base1a, base2a),
                                       (a1b, a2b, base1b, base2b)):
                cnt1 = a1 - (b1o - 1)
                cnt2 = a2 - (b2o - 1)
                b1v[pl.ds(b1o, 16)] = (lane < cnt1).astype(jnp.float32)
                b1v[pl.ds(b1o + 16, 16)] = ((lane + 16) < cnt1).astype(
                    jnp.float32)
                b2v[pl.ds(b2o, 16)] = (lane < cnt2).astype(jnp.float32)
                b2v[pl.ds(b2o + 16, 16)] = ((lane + 16) < cnt2).astype(
                    jnp.float32)
                b2v[pl.ds(b2o + 32, 16)] = ((lane + 32) < cnt2).astype(
                    jnp.float32)
            return carry_unused

        lax.fori_loop(0, _QPW // 2, per_pair, jnp.int32(0))
        for ch, b in enumerate((b1x, b1y, b1z, b1v)):
            pltpu.sync_copy(b, out1.at[pl.ds(((st * 4 + ch) * N + q0) * P1,
                                             _QPW * P1)])
        for ch, b in enumerate((b2x, b2y, b2z, b2v)):
            pltpu.sync_copy(b, out2.at[pl.ds(((st * 4 + ch) * N + q0) * P2,
                                             _QPW * P2)])


@functools.lru_cache(maxsize=1)
def _bq_sc_build():
    return pl.kernel(
        _bq_sc_body,
        out_type=(
            jax.ShapeDtypeStruct((NST * 4 * N * P1,), jnp.float32),
            jax.ShapeDtypeStruct((NST * 4 * N * P2,), jnp.float32),
        ),
        mesh=plsc.VectorSubcoreMesh(core_axis_name="c", subcore_axis_name="s"),
        compiler_params=pltpu.CompilerParams(needs_layout_passes=False),
        scratch_types=[
            pltpu.VMEM((3 * 3 * N,), jnp.float32),
            pltpu.VMEM((_QPW * P1,), jnp.float32),
            pltpu.VMEM((_QPW * P1,), jnp.float32),
            pltpu.VMEM((_QPW * P1,), jnp.float32),
            pltpu.VMEM((_QPW * P1,), jnp.float32),
            pltpu.VMEM((_QPW * P2,), jnp.float32),
            pltpu.VMEM((_QPW * P2,), jnp.float32),
            pltpu.VMEM((_QPW * P2,), jnp.float32),
            pltpu.VMEM((_QPW * P2,), jnp.float32),
        ],
    )


_QB = 256
_QSET = [qs for qs, _ in _STACK_PAIRS]


def _mlp_tc_body(rel1, rel2, tg, ts, tv, w1, b1, w2, b2, wo, bo,
                 og, osf, ov):
    st = pl.program_id(1)
    wv = (lax.broadcasted_iota(jnp.int32, (4, BQH), 0) == 3).astype(jnp.float32)
    feats = []
    for sc, P in enumerate((P1, P2)):
        rel = (rel1 if sc == 0 else rel2)[0]
        dn = (((0,), (0,)), ((), ()))
        h = lax.dot_general(rel, w1[0, sc], dn,
                            preferred_element_type=jnp.float32)
        h = jnp.maximum(h + b1[0, sc][None, :], 0.0)
        h = jnp.dot(h, w2[0, sc], preferred_element_type=jnp.float32)
        h = jnp.maximum(h + b2[0, sc][None, :], 0.0)
        v = lax.dot_general(rel, wv, dn,
                            preferred_element_type=jnp.float32)
        h = (h * v).reshape(_QB, P, BQH)
        feats.append(jnp.max(h, axis=1))
    f = jnp.concatenate(feats, axis=-1)
    contrib = jnp.dot(f, wo[0], preferred_element_type=jnp.float32)
    contrib = contrib + bo[0]

    for out_ref, tok_ref, qset in ((og, tg, 0), (osf, ts, 1), (ov, tv, 2)):
        first = [s for s, q in enumerate(_QSET) if q == qset][0]
        rest = [s for s, q in enumerate(_QSET) if q == qset][1:]

        @pl.when(st == first)
        def _():
            out_ref[...] = tok_ref[...] + contrib

        for s in rest:
            @pl.when(st == s)
            def _():
                out_ref[...] = out_ref[...] + contrib


def _mlp_tc(rel4_1, rel4_2, tg, ts, tv, W1e, b1s, W2s, b2s, WOs, BOs):
    grid = (N // _QB, NST)
    tok_spec = pl.BlockSpec((_QB, HID), lambda i, st: (i, 0))
    return pl.pallas_call(
        _mlp_tc_body,
        grid=grid,
        in_specs=[
            pl.BlockSpec((1, 4, _QB * P1), lambda i, st: (st, 0, i)),
            pl.BlockSpec((1, 4, _QB * P2), lambda i, st: (st, 0, i)),
            tok_spec, tok_spec, tok_spec,
            pl.BlockSpec((1, 2, 4, BQH), lambda i, st: (st, 0, 0, 0)),
            pl.BlockSpec((1, 2, BQH), lambda i, st: (st, 0, 0)),
            pl.BlockSpec((1, 2, BQH, BQH), lambda i, st: (st, 0, 0, 0)),
            pl.BlockSpec((1, 2, BQH), lambda i, st: (st, 0, 0)),
            pl.BlockSpec((1, 2 * BQH, HID), lambda i, st: (st, 0, 0)),
            pl.BlockSpec((1, 1, HID), lambda i, st: (st, 0, 0)),
        ],
        out_specs=[tok_spec, tok_spec, tok_spec],
        out_shape=[jax.ShapeDtypeStruct((N, HID), jnp.float32)] * 3,
    )(rel4_1, rel4_2, tg, ts, tv, W1e, b1s, W2s, b2s,
      WOs, BOs.reshape(NST, 1, HID))


def kernel(geometry_points, surface_points, volume_points, geo_tokens,
           surf_tokens, vol_tokens, params):
    pts = jnp.stack([geometry_points[0].T, surface_points[0].T,
                     volume_points[0].T])
    o1, o2 = _bq_sc_build()(pts.reshape(-1))
    rel4_1 = o1.reshape(NST, 4, N * P1)
    rel4_2 = o2.reshape(NST, 4, N * P2)

    W1s = jnp.stack([jnp.stack([params[n]["scales"][i]["W1"] for i in range(2)])
                     for n in _STACK_NAMES])
    W1e = jnp.concatenate(
        [W1s, jnp.zeros((NST, 2, 1, BQH), jnp.float32)], axis=2)
    b1s = jnp.stack([jnp.stack([params[n]["scales"][i]["b1"] for i in range(2)])
                     for n in _STACK_NAMES])
    W2s = jnp.stack([jnp.stack([params[n]["scales"][i]["W2"] for i in range(2)])
                     for n in _STACK_NAMES])
    b2s = jnp.stack([jnp.stack([params[n]["scales"][i]["b2"] for i in range(2)])
                     for n in _STACK_NAMES])
    WOs = jnp.stack([params[n]["Wout"] for n in _STACK_NAMES])
    BOs = jnp.stack([params[n]["bout"] for n in _STACK_NAMES])

    geo, surf, vol = _mlp_tc(rel4_1, rel4_2, geo_tokens[0], surf_tokens[0],
                             vol_tokens[0], W1e, b1s, W2s, b2s, WOs, BOs)
    return geo[None], surf[None], vol[None]
